# baseline (speedup 1.0000x reference)
import jax
import jax.numpy as jnp
from jax import lax
from jax.experimental import pallas as pl

B, D1, D2 = 64, 2048, 512
SG = 8
NG = D1 // SG


def _tc_body(x_ref, o_ref):
    def group_body(i, carry):
        mv, mi = carry
        v = x_ref[0, pl.ds(i * SG, SG), :]
        lt = v < mv
        gi = jnp.full((SG, D2), i, jnp.int32)
        return jnp.where(lt, v, mv), jnp.where(lt, gi, mi)

    mv0 = jnp.full((SG, D2), jnp.inf, jnp.float32)
    mi0 = jnp.zeros((SG, D2), jnp.int32)
    mv, mi = lax.fori_loop(0, NG, group_body, (mv0, mi0))

    sub = lax.broadcasted_iota(jnp.int32, (SG, D2), 0)
    row = mi * SG + sub

    for sh in (4, 2, 1):
        mv2 = pltpu_roll(mv, sh)
        row2 = pltpu_roll(row, sh)
        take = (mv2 < mv) | ((mv2 == mv) & (row2 < row))
        mv = jnp.where(take, mv2, mv)
        row = jnp.where(take, row2, row)

    o_ref[0, 0] = row[0, :]


def pltpu_roll(x, shift):
    return jnp.roll(x, -shift, axis=0)


_tc_argmin = pl.pallas_call(
    _tc_body,
    grid=(B,),
    in_specs=[pl.BlockSpec((1, D1, D2), lambda i: (i, 0, 0))],
    out_specs=pl.BlockSpec((1, 1, D2), lambda i: (i, 0, 0)),
    out_shape=jax.ShapeDtypeStruct((B, 1, D2), jnp.int32),
)


@jax.jit
def kernel(x):
    return _tc_argmin(x)[:, 0, :].astype(jnp.int64)


# speedup vs baseline: 1.2693x; 1.2693x over previous
import jax
import jax.numpy as jnp
from jax import lax
from jax.experimental import pallas as pl

B, D1, D2 = 64, 2048, 512
SG = 32
NG = D1 // SG


def _tc_body(x_ref, o_ref):
    def group_body(i, carry):
        mv, mi = carry
        v = x_ref[0, pl.ds(i * SG, SG), :]
        lt = v < mv
        gi = jnp.full((SG, D2), i, jnp.int32)
        return jnp.where(lt, v, mv), jnp.where(lt, gi, mi)

    mv0 = jnp.full((SG, D2), jnp.inf, jnp.float32)
    mi0 = jnp.zeros((SG, D2), jnp.int32)
    mv, mi = lax.fori_loop(0, NG, group_body, (mv0, mi0), unroll=4)

    sub = lax.broadcasted_iota(jnp.int32, (SG, D2), 0)
    row = mi * SG + sub

    sh = SG // 2
    while sh >= 1:
        mv2 = jnp.roll(mv, -sh, axis=0)
        row2 = jnp.roll(row, -sh, axis=0)
        take = (mv2 < mv) | ((mv2 == mv) & (row2 < row))
        mv = jnp.where(take, mv2, mv)
        row = jnp.where(take, row2, row)
        sh //= 2

    o_ref[0, 0] = row[0, :]


_tc_argmin = pl.pallas_call(
    _tc_body,
    grid=(B,),
    in_specs=[pl.BlockSpec((1, D1, D2), lambda i: (i, 0, 0))],
    out_specs=pl.BlockSpec((1, 1, D2), lambda i: (i, 0, 0)),
    out_shape=jax.ShapeDtypeStruct((B, 1, D2), jnp.int32),
)


@jax.jit
def kernel(x):
    return _tc_argmin(x)[:, 0, :].astype(jnp.int64)
